# Initial kernel scaffold; baseline (speedup 1.0000x reference)
import jax
import jax.numpy as jnp
from jax.experimental import pallas as pl


def kernel(x, W_attn, W_proj):
    raise NotImplementedError("write your pallas kernel here")


# speedup vs baseline: 1.3188x; 1.3188x over previous
import functools

import jax
import jax.numpy as jnp
import numpy as np
from jax.experimental import pallas as pl
from jax.experimental.pallas import tpu as pltpu

N_HEAD = 16
K_KEEP = 512
RECENCY = 64


def _qkv_kernel(x_ref, w_ref, o_ref, *, hpb, dh):
    r = jax.lax.dot_general(
        x_ref[...], w_ref[...],
        (((1,), (0,)), ((), ())),
        preferred_element_type=jnp.float32)
    r = r.astype(jnp.bfloat16)
    for ih in range(hpb):
        o_ref[ih, 0, :, :] = r[:, ih * dh:(ih + 1) * dh]


def _importance_kernel(q_ref, k_ref, bias_ref, colsum_ref, *,
                       t, bq, scale, k_keep, recency):
    qi = pl.program_id(2)

    @pl.when(qi == 0)
    def _():
        colsum_ref[...] = jnp.zeros_like(colsum_ref)

    q = q_ref[0, 0, :, :]
    k = k_ref[0, 0, :, :]
    s = jax.lax.dot_general(
        q, k, (((1,), (1,)), ((), ())),
        preferred_element_type=jnp.float32) * scale
    row = qi * bq + jax.lax.broadcasted_iota(jnp.int32, (bq, t), 0)
    col = jax.lax.broadcasted_iota(jnp.int32, (bq, t), 1)
    s = jnp.where(col <= row, s, -jnp.inf)
    m = jnp.max(s, axis=1, keepdims=True)
    p = jnp.exp(s - m)
    l = jnp.sum(p, axis=1, keepdims=True)
    colsum_ref[...] += jnp.sum(p / l, axis=0, keepdims=True)

    @pl.when(qi == pl.num_programs(2) - 1)
    def _():
        v = colsum_ref[...] * (1.0 / t)
        cidx = jax.lax.broadcasted_iota(jnp.int32, (1, t), 1)
        v = jnp.where(cidx >= t - recency, 1.0, v)
        vb = jax.lax.bitcast_convert_type(v, jnp.int32)

        def body(i, tb):
            cand = tb | (1 << (30 - i))
            cnt = jnp.sum((vb >= cand).astype(jnp.int32))
            return jnp.where(cnt >= k_keep, cand, tb)

        tbits = jax.lax.fori_loop(0, 31, body, jnp.int32(0))
        bias_ref[0, :, :] = jnp.where(vb >= tbits, 0.0, -jnp.inf)


def _pruned_attn_kernel(q_ref, k_ref, v_ref, bias_ref, o_ref, *, scale):
    q = q_ref[0, 0, :, :]
    k = k_ref[0, 0, :, :]
    v = v_ref[0, 0, :, :]
    s = jax.lax.dot_general(
        q, k, (((1,), (1,)), ((), ())),
        preferred_element_type=jnp.float32) * scale + bias_ref[0, :, :]
    m = jnp.max(s, axis=1, keepdims=True)
    p = jnp.exp(s - m)
    l = jnp.sum(p, axis=1, keepdims=True)
    o = jax.lax.dot_general(
        p.astype(jnp.bfloat16), v, (((1,), (0,)), ((), ())),
        preferred_element_type=jnp.float32) / l
    o_ref[0, 0, :, :] = o.astype(jnp.bfloat16)


def _proj_kernel(x_ref, w_ref, o_ref, acc_ref):
    h = pl.program_id(2)

    @pl.when(h == 0)
    def _():
        acc_ref[...] = jnp.zeros_like(acc_ref)

    acc_ref[...] += jax.lax.dot_general(
        x_ref[0, 0, :, :], w_ref[0, :, :],
        (((1,), (0,)), ((), ())),
        preferred_element_type=jnp.float32)

    @pl.when(h == pl.num_programs(2) - 1)
    def _():
        o_ref[0, :, :] = acc_ref[...]


def _forward(x, w_attn, w_proj, n_head, k_keep, recency, bq,
             interpret=False):
    b, t, c = x.shape
    dh = c // n_head
    scale = np.float32(1.0 / np.sqrt(dh))
    nq = t // bq
    bm = bq
    nm = t // bm

    x2 = x.reshape(b * t, c).astype(jnp.bfloat16)
    w_attn = w_attn.astype(jnp.bfloat16)
    bn = int(np.gcd(8 * dh, 3 * c))
    hpb = bn // dh
    qkv = pl.pallas_call(
        functools.partial(_qkv_kernel, hpb=hpb, dh=dh),
        grid=(b * t // bm, 3 * c // bn),
        in_specs=[
            pl.BlockSpec((bm, c), lambda i, j: (i, 0)),
            pl.BlockSpec((c, bn), lambda i, j: (0, j)),
        ],
        out_specs=pl.BlockSpec(
            (hpb, 1, bm, dh),
            lambda i, j, _nm=nm: (j, i // _nm, i % _nm, 0)),
        out_shape=jax.ShapeDtypeStruct((3 * n_head, b, t, dh), jnp.bfloat16),
        interpret=interpret,
    )(x2, w_attn)

    grid = (b, n_head, nq)
    q_spec = pl.BlockSpec((1, 1, bq, dh), lambda bi, h, qi: (h, bi, qi, 0))
    k_spec = pl.BlockSpec((1, 1, t, dh),
                          lambda bi, h, qi: (h + n_head, bi, 0, 0))
    v_spec = pl.BlockSpec((1, 1, t, dh),
                          lambda bi, h, qi: (h + 2 * n_head, bi, 0, 0))
    bias_spec = pl.BlockSpec((1, 1, t),
                             lambda bi, h, qi: (bi * n_head + h, 0, 0))

    bias = pl.pallas_call(
        functools.partial(_importance_kernel, t=t, bq=bq, scale=scale,
                          k_keep=k_keep, recency=recency),
        grid=grid,
        in_specs=[q_spec, k_spec],
        out_specs=bias_spec,
        out_shape=jax.ShapeDtypeStruct((b * n_head, 1, t), jnp.float32),
        scratch_shapes=[pltpu.VMEM((1, t), jnp.float32)],
        interpret=interpret,
    )(qkv, qkv)

    out_heads = pl.pallas_call(
        functools.partial(_pruned_attn_kernel, scale=scale),
        grid=grid,
        in_specs=[q_spec, k_spec, v_spec, bias_spec],
        out_specs=pl.BlockSpec((1, 1, bq, dh),
                               lambda bi, h, qi: (h, bi, qi, 0)),
        out_shape=jax.ShapeDtypeStruct((n_head, b, t, dh), jnp.bfloat16),
        interpret=interpret,
    )(qkv, qkv, qkv, bias)

    w3 = w_proj.reshape(n_head, dh, c).astype(jnp.bfloat16)
    out = pl.pallas_call(
        _proj_kernel,
        grid=(b, nm, n_head),
        in_specs=[
            pl.BlockSpec((1, 1, bm, dh), lambda bi, mi, h: (h, bi, mi, 0)),
            pl.BlockSpec((1, dh, c), lambda bi, mi, h: (h, 0, 0)),
        ],
        out_specs=pl.BlockSpec((1, bm, c), lambda bi, mi, h: (bi, mi, 0)),
        out_shape=jax.ShapeDtypeStruct((b, t, c), jnp.float32),
        scratch_shapes=[pltpu.VMEM((bm, c), jnp.float32)],
        interpret=interpret,
    )(out_heads, w3)
    return out


def kernel(x, W_attn, W_proj):
    return _forward(x, W_attn, W_proj, n_head=N_HEAD, k_keep=K_KEEP,
                    recency=RECENCY, bq=256)
